# speedup vs baseline: 1.0221x; 1.0221x over previous
import functools

import jax
import jax.numpy as jnp
from jax.experimental import pallas as pl


def _gcn_block(adj_ref, x_ref, w_ref, out_ref):
    tmp = jnp.dot(
        adj_ref[...].astype(jnp.bfloat16),
        x_ref[...],
        preferred_element_type=jnp.float32,
    )
    out_ref[...] = jnp.dot(
        tmp.astype(jnp.bfloat16),
        w_ref[...],
        preferred_element_type=jnp.float32,
    )


@functools.partial(jax.jit, static_argnames=("block_m",))
def _gcn(adj, x, W, block_m=400):
    m, k = adj.shape
    d_in, d_out = W.shape
    bm = min(block_m, m)
    return pl.pallas_call(
        _gcn_block,
        grid=(pl.cdiv(m, bm),),
        in_specs=[
            pl.BlockSpec((bm, k), lambda i: (i, 0)),
            pl.BlockSpec((k, d_in), lambda i: (0, 0)),
            pl.BlockSpec((d_in, d_out), lambda i: (0, 0)),
        ],
        out_specs=pl.BlockSpec((bm, d_out), lambda i: (i, 0)),
        out_shape=jax.ShapeDtypeStruct((m, d_out), jnp.float32),
    )(adj, x.astype(jnp.bfloat16), W.astype(jnp.bfloat16))


def kernel(adj, x, W):
    return _gcn(adj, x, W)
